# baseline (speedup 1.0000x reference)
import functools

import jax
import jax.numpy as jnp
from jax import lax
from jax.experimental import pallas as pl
from jax.experimental.pallas import tpu as pltpu
from jax.experimental.pallas import tpu_sc as plsc

N_USERS = 5000
N_ITEMS = 5000
D = 128
N_NODES = N_USERS + N_ITEMS
N_LAYER = 3
N_EDGE = 320000
BATCH = 4096

NC = 2
NS = 16
NW = NC * NS
CHUNK = 112
CH_PER_W = 90
CH0 = 60
CH1 = 120
E_PAD = NW * CH_PER_W * CHUNK
N_NODES_PAD = 10240
ROWS_PER_SUB = N_NODES_PAD // NS
BPW = BATCH // NW

_mesh = plsc.VectorSubcoreMesh(core_axis_name="c", subcore_axis_name="s")


NROW = 3
NIDX = 6


@functools.partial(
    pl.kernel,
    out_type=jax.ShapeDtypeStruct((NC, N_NODES_PAD, D), jnp.float32),
    mesh=_mesh,
    scratch_types=(
        [pltpu.VMEM((2, CHUNK), jnp.int32) for _ in range(NIDX)]
        + [pltpu.VMEM((CHUNK,), jnp.float32) for _ in range(NIDX)]
        + [pltpu.VMEM((CHUNK, D), jnp.float32) for _ in range(NROW)]
        + [pltpu.SemaphoreType.DMA] * (NIDX + 2 * NROW)
        + [pltpu.VMEM_SHARED((N_NODES_PAD, D), jnp.float32)]
    ),
)
def _sc_propagate(pref_hbm, epk_hbm, w_hbm, zeros_hbm, out_hbm, *refs):
    ebuf = refs[0:NIDX]
    wbuf = refs[NIDX:2 * NIDX]
    rows = refs[2 * NIDX:2 * NIDX + NROW]
    isem = refs[2 * NIDX + NROW:3 * NIDX + NROW]
    gsem = refs[3 * NIDX + NROW:3 * NIDX + 2 * NROW]
    ssem = refs[3 * NIDX + 2 * NROW:3 * NIDX + 3 * NROW]
    acc_sh = refs[3 * NIDX + 3 * NROW]
    c = lax.axis_index("c")
    s = lax.axis_index("s")
    ch_c = jnp.where(c == 0, CH0, CH1)
    base = jnp.where(c == 0, s * CH0, NS * CH0 + s * CH1)
    pltpu.sync_copy(zeros_hbm.at[pl.ds(s * ROWS_PER_SUB, ROWS_PER_SUB)],
                    acc_sh.at[pl.ds(s * ROWS_PER_SUB, ROWS_PER_SUB)])
    plsc.subcore_barrier()

    def _issue_idx(ci, eb):
        pltpu.async_copy(epk_hbm.at[base + ci], ebuf[eb], isem[eb])
        pltpu.async_copy(w_hbm.at[base + ci], wbuf[eb], isem[eb])

    def _wait_idx(eb):
        pltpu.make_async_copy(epk_hbm.at[base], ebuf[eb], isem[eb]).wait()
        pltpu.make_async_copy(w_hbm.at[base], wbuf[eb], isem[eb]).wait()

    def _issue_gather(eb, r):
        pltpu.async_copy(pref_hbm.at[ebuf[eb].at[0]], rows[r], gsem[r])

    def _wait_gather(eb, r):
        pltpu.make_async_copy(pref_hbm.at[ebuf[eb].at[0]], rows[r], gsem[r]).wait()

    def _issue_scatter(eb, r):
        pltpu.async_copy(rows[r], acc_sh.at[ebuf[eb].at[1]], ssem[r], add=True)

    def _wait_scatter(eb, r):
        pltpu.make_async_copy(rows[r], acc_sh.at[ebuf[eb].at[1]], ssem[r]).wait()

    def _scale(eb, r):
        def grp(g, carry):
            wreg = wbuf[eb][pl.ds(g * 16, 16)]
            for i in range(16):
                wb = jnp.full((16,), wreg[i], jnp.float32)
                e = g * 16 + i
                for j in range(D // 16):
                    rows[r][e, pl.ds(j * 16, 16)] = rows[r][e, pl.ds(j * 16, 16)] * wb
            return carry
        lax.fori_loop(0, CHUNK // 16, grp, 0)

    _issue_idx(0, 0)
    _issue_idx(1, 1)
    _issue_idx(2, 2)
    _wait_idx(0)
    _issue_gather(0, 0)

    def outer(k, carry):
        for u in range(NIDX):
            ci = NIDX * k + u
            r = u % NROW

            @pl.when(ci >= 2)
            def _():
                _wait_scatter((u + 4) % NIDX, (u + 1) % NROW)

            @pl.when(ci + 1 < ch_c)
            def _():
                _wait_idx((u + 1) % NIDX)
                _issue_gather((u + 1) % NIDX, (u + 1) % NROW)

            @pl.when(ci + 3 < ch_c)
            def _():
                _issue_idx(ci + 3, (u + 3) % NIDX)

            _wait_gather(u, r)
            _scale(u, r)
            _issue_scatter(u, r)
        return carry

    lax.fori_loop(0, ch_c // NIDX, outer, 0)
    _wait_scatter(4, 1)
    _wait_scatter(5, 2)
    plsc.subcore_barrier()
    pltpu.sync_copy(acc_sh.at[pl.ds(s * ROWS_PER_SUB, ROWS_PER_SUB)],
                    out_hbm.at[c, pl.ds(s * ROWS_PER_SUB, ROWS_PER_SUB)])


def _leaky_norm(x):
    y = jnp.where(x >= 0, x, 0.1 * x)
    n = jnp.sqrt(jnp.sum(y * y, axis=-1, keepdims=True))
    return y / jnp.maximum(n, 1e-12)


def _tc_prep_body(cat_ref, out_ref):
    out_ref[...] = _leaky_norm(cat_ref[...])


_tc_prep = pl.pallas_call(
    _tc_prep_body,
    out_shape=jax.ShapeDtypeStruct((N_NODES, D), jnp.float32),
    grid=(10,),
    in_specs=[pl.BlockSpec((1000, D), lambda i: (i, 0))],
    out_specs=pl.BlockSpec((1000, D), lambda i: (i, 0)),
)


def _tc_post_body(part_ref, macc_ref, pref_out, macc_out):
    p = _leaky_norm(part_ref[0] + part_ref[1])
    pref_out[...] = p
    macc_out[...] = macc_ref[...] + p


_tc_post = pl.pallas_call(
    _tc_post_body,
    out_shape=[jax.ShapeDtypeStruct((N_NODES, D), jnp.float32)] * 2,
    grid=(10,),
    in_specs=[pl.BlockSpec((NC, 1000, D), lambda i: (0, i, 0)),
              pl.BlockSpec((1000, D), lambda i: (i, 0))],
    out_specs=[pl.BlockSpec((1000, D), lambda i: (i, 0))] * 2,
)


@functools.partial(
    pl.kernel,
    out_type=jax.ShapeDtypeStruct((8, BATCH, D), jnp.float32),
    mesh=_mesh,
    scratch_types=[
        pltpu.VMEM((BPW,), jnp.int32),
        pltpu.VMEM((BPW, D), jnp.float32),
        pltpu.SemaphoreType.DMA,
    ],
)
def _sc_gather(upref, ipref, ustr, istr, users_i, adj_i, weak_i, strong_i,
               out_hbm, idx_v, rows_v, sem):
    c = lax.axis_index("c")
    s = lax.axis_index("s")
    wid = c * NS + s
    tasks = ((upref, users_i), (ipref, adj_i), (ipref, weak_i), (ipref, strong_i),
             (ustr, users_i), (istr, adj_i), (istr, weak_i), (istr, strong_i))
    for t, (tab, idx) in enumerate(tasks):
        pltpu.sync_copy(idx.at[wid], idx_v)
        pltpu.async_copy(tab.at[idx_v], rows_v, sem).wait()
        pltpu.sync_copy(rows_v, out_hbm.at[t, pl.ds(wid * BPW, BPW)])


def _tc_batch_body(g_ref, out_ref):
    up = g_ref[0] * 0.25
    ipa = g_ref[1] * 0.25
    ipw = g_ref[2] * 0.25
    ips = g_ref[3] * 0.25
    us = g_ref[4]

    def _dot(a, b):
        return jnp.sum(a * b, axis=-1)

    def _norm(x):
        n = jnp.sqrt(jnp.sum(x * x, axis=-1, keepdims=True))
        return x / jnp.maximum(n, 1e-12)

    usn = _norm(us)

    def _gs(im):
        imn = _norm(im)
        d = jnp.sqrt(jnp.sum((usn - imn) ** 2, axis=-1) + 1e-12)
        return (2.0 - d) * 0.5

    ga = jax.nn.sigmoid(_dot(up, ipa)) * _gs(g_ref[5])
    gw = jax.nn.sigmoid(_dot(up, ipw)) * _gs(g_ref[6])
    gst = jax.nn.sigmoid(_dot(up, ips)) * _gs(g_ref[7])
    out_ref[...] = jnp.stack([ga, gw, gst], axis=0)


_tc_batch = pl.pallas_call(
    _tc_batch_body,
    out_shape=jax.ShapeDtypeStruct((3, BATCH), jnp.float32),
    grid=(8,),
    in_specs=[pl.BlockSpec((8, 512, D), lambda i: (0, i, 0))],
    out_specs=pl.BlockSpec((3, 512), lambda i: (0, i)),
)


def kernel(users, adjacent_items, weak_items, strong_items, edge_index, edge_weight,
           user_preference, item_preference, user_structure, item_structure):
    dst = edge_index[0].astype(jnp.int32)
    src = edge_index[1].astype(jnp.int32)
    w = edge_weight.astype(jnp.float32)
    pad = E_PAD - N_EDGE
    src_p = jnp.concatenate([src, jnp.zeros((pad,), jnp.int32)]).reshape(NW * CH_PER_W, CHUNK)
    dst_p = jnp.concatenate([dst, jnp.zeros((pad,), jnp.int32)]).reshape(NW * CH_PER_W, CHUNK)
    w_p = jnp.concatenate([w, jnp.zeros((pad,), jnp.float32)]).reshape(NW * CH_PER_W, CHUNK)
    epk = jnp.stack([src_p, dst_p], axis=1)
    zeros = jnp.zeros((N_NODES_PAD, D), jnp.float32)

    cat = jnp.concatenate([user_preference, item_preference], axis=0)
    pref = _tc_prep(cat)
    macc = pref
    for _ in range(N_LAYER):
        part = _sc_propagate(pref, epk, w_p, zeros)[:, :N_NODES, :]
        pref, macc = _tc_post(part, macc)

    users_pref = macc[:N_USERS]
    items_pref = macc[N_USERS:]
    ui = users.astype(jnp.int32).reshape(NW, BPW)
    ai = adjacent_items.astype(jnp.int32).reshape(NW, BPW)
    wi = weak_items.astype(jnp.int32).reshape(NW, BPW)
    si = strong_items.astype(jnp.int32).reshape(NW, BPW)
    g8 = _sc_gather(users_pref, items_pref, user_structure, item_structure,
                    ui, ai, wi, si)
    return _tc_batch(g8)


# speedup vs baseline: 1.1883x; 1.1883x over previous
import functools

import jax
import jax.numpy as jnp
from jax import lax
from jax.experimental import pallas as pl
from jax.experimental.pallas import tpu as pltpu
from jax.experimental.pallas import tpu_sc as plsc

N_USERS = 5000
N_ITEMS = 5000
D = 128
N_NODES = N_USERS + N_ITEMS
N_LAYER = 3
N_EDGE = 320000
BATCH = 4096

NC = 2
NS = 16
NW = NC * NS
CHUNK = 112
CH_PER_W = 90
CH0 = 120
CH1 = 60
E_PAD = NW * CH_PER_W * CHUNK
N_NODES_PAD = 10240
ROWS_PER_SUB = N_NODES_PAD // NS
BPW = BATCH // NW

_mesh = plsc.VectorSubcoreMesh(core_axis_name="c", subcore_axis_name="s")


NROW = 3
NIDX = 6


@functools.partial(
    pl.kernel,
    out_type=jax.ShapeDtypeStruct((NC, N_NODES_PAD, D), jnp.float32),
    mesh=_mesh,
    scratch_types=(
        [pltpu.VMEM((2, CHUNK), jnp.int32) for _ in range(NIDX)]
        + [pltpu.VMEM((CHUNK,), jnp.float32) for _ in range(NIDX)]
        + [pltpu.VMEM((CHUNK, D), jnp.float32) for _ in range(NROW)]
        + [pltpu.SemaphoreType.DMA] * (NIDX + 2 * NROW)
        + [pltpu.VMEM_SHARED((N_NODES_PAD, D), jnp.float32)]
    ),
)
def _sc_propagate(pref_hbm, epk_hbm, w_hbm, zeros_hbm, out_hbm, *refs):
    ebuf = refs[0:NIDX]
    wbuf = refs[NIDX:2 * NIDX]
    rows = refs[2 * NIDX:2 * NIDX + NROW]
    isem = refs[2 * NIDX + NROW:3 * NIDX + NROW]
    gsem = refs[3 * NIDX + NROW:3 * NIDX + 2 * NROW]
    ssem = refs[3 * NIDX + 2 * NROW:3 * NIDX + 3 * NROW]
    acc_sh = refs[3 * NIDX + 3 * NROW]
    c = lax.axis_index("c")
    s = lax.axis_index("s")
    ch_c = jnp.where(c == 0, CH0, CH1)
    base = jnp.where(c == 0, s * CH0, NS * CH0 + s * CH1)
    pltpu.sync_copy(zeros_hbm.at[pl.ds(s * ROWS_PER_SUB, ROWS_PER_SUB)],
                    acc_sh.at[pl.ds(s * ROWS_PER_SUB, ROWS_PER_SUB)])
    plsc.subcore_barrier()

    def _issue_idx(ci, eb):
        pltpu.async_copy(epk_hbm.at[base + ci], ebuf[eb], isem[eb])
        pltpu.async_copy(w_hbm.at[base + ci], wbuf[eb], isem[eb])

    def _wait_idx(eb):
        pltpu.make_async_copy(epk_hbm.at[base], ebuf[eb], isem[eb]).wait()
        pltpu.make_async_copy(w_hbm.at[base], wbuf[eb], isem[eb]).wait()

    def _issue_gather(eb, r):
        pltpu.async_copy(pref_hbm.at[ebuf[eb].at[0]], rows[r], gsem[r])

    def _wait_gather(eb, r):
        pltpu.make_async_copy(pref_hbm.at[ebuf[eb].at[0]], rows[r], gsem[r]).wait()

    def _issue_scatter(eb, r):
        pltpu.async_copy(rows[r], acc_sh.at[ebuf[eb].at[1]], ssem[r], add=True)

    def _wait_scatter(eb, r):
        pltpu.make_async_copy(rows[r], acc_sh.at[ebuf[eb].at[1]], ssem[r]).wait()

    def _scale(eb, r):
        def grp(g, carry):
            wreg = wbuf[eb][pl.ds(g * 16, 16)]
            for i in range(16):
                wb = jnp.full((16,), wreg[i], jnp.float32)
                e = g * 16 + i
                for j in range(D // 16):
                    rows[r][e, pl.ds(j * 16, 16)] = rows[r][e, pl.ds(j * 16, 16)] * wb
            return carry
        lax.fori_loop(0, CHUNK // 16, grp, 0)

    _issue_idx(0, 0)
    _issue_idx(1, 1)
    _issue_idx(2, 2)
    _wait_idx(0)
    _issue_gather(0, 0)

    def outer(k, carry):
        for u in range(NIDX):
            ci = NIDX * k + u
            r = u % NROW

            @pl.when(ci >= 2)
            def _():
                _wait_scatter((u + 4) % NIDX, (u + 1) % NROW)

            @pl.when(ci + 1 < ch_c)
            def _():
                _wait_idx((u + 1) % NIDX)
                _issue_gather((u + 1) % NIDX, (u + 1) % NROW)

            @pl.when(ci + 3 < ch_c)
            def _():
                _issue_idx(ci + 3, (u + 3) % NIDX)

            _wait_gather(u, r)
            _scale(u, r)
            _issue_scatter(u, r)
        return carry

    lax.fori_loop(0, ch_c // NIDX, outer, 0)
    _wait_scatter(4, 1)
    _wait_scatter(5, 2)
    plsc.subcore_barrier()
    pltpu.sync_copy(acc_sh.at[pl.ds(s * ROWS_PER_SUB, ROWS_PER_SUB)],
                    out_hbm.at[c, pl.ds(s * ROWS_PER_SUB, ROWS_PER_SUB)])


def _leaky_norm(x):
    y = jnp.where(x >= 0, x, 0.1 * x)
    n = jnp.sqrt(jnp.sum(y * y, axis=-1, keepdims=True))
    return y / jnp.maximum(n, 1e-12)


def _tc_prep_body(cat_ref, out_ref):
    out_ref[...] = _leaky_norm(cat_ref[...])


_tc_prep = pl.pallas_call(
    _tc_prep_body,
    out_shape=jax.ShapeDtypeStruct((N_NODES, D), jnp.float32),
    grid=(10,),
    in_specs=[pl.BlockSpec((1000, D), lambda i: (i, 0))],
    out_specs=pl.BlockSpec((1000, D), lambda i: (i, 0)),
)


def _tc_post_body(part_ref, macc_ref, pref_out, macc_out):
    p = _leaky_norm(part_ref[0] + part_ref[1])
    pref_out[...] = p
    macc_out[...] = macc_ref[...] + p


_tc_post = pl.pallas_call(
    _tc_post_body,
    out_shape=[jax.ShapeDtypeStruct((N_NODES, D), jnp.float32)] * 2,
    grid=(10,),
    in_specs=[pl.BlockSpec((NC, 1000, D), lambda i: (0, i, 0)),
              pl.BlockSpec((1000, D), lambda i: (i, 0))],
    out_specs=[pl.BlockSpec((1000, D), lambda i: (i, 0))] * 2,
)


@functools.partial(
    pl.kernel,
    out_type=jax.ShapeDtypeStruct((8, BATCH, D), jnp.float32),
    mesh=_mesh,
    scratch_types=[
        pltpu.VMEM((BPW,), jnp.int32),
        pltpu.VMEM((BPW, D), jnp.float32),
        pltpu.SemaphoreType.DMA,
    ],
)
def _sc_gather(upref, ipref, ustr, istr, users_i, adj_i, weak_i, strong_i,
               out_hbm, idx_v, rows_v, sem):
    c = lax.axis_index("c")
    s = lax.axis_index("s")
    wid = c * NS + s
    tasks = ((upref, users_i), (ipref, adj_i), (ipref, weak_i), (ipref, strong_i),
             (ustr, users_i), (istr, adj_i), (istr, weak_i), (istr, strong_i))
    for t, (tab, idx) in enumerate(tasks):
        pltpu.sync_copy(idx.at[wid], idx_v)
        pltpu.async_copy(tab.at[idx_v], rows_v, sem).wait()
        pltpu.sync_copy(rows_v, out_hbm.at[t, pl.ds(wid * BPW, BPW)])


def _tc_batch_body(g_ref, out_ref):
    up = g_ref[0] * 0.25
    ipa = g_ref[1] * 0.25
    ipw = g_ref[2] * 0.25
    ips = g_ref[3] * 0.25
    us = g_ref[4]

    def _dot(a, b):
        return jnp.sum(a * b, axis=-1)

    def _norm(x):
        n = jnp.sqrt(jnp.sum(x * x, axis=-1, keepdims=True))
        return x / jnp.maximum(n, 1e-12)

    usn = _norm(us)

    def _gs(im):
        imn = _norm(im)
        d = jnp.sqrt(jnp.sum((usn - imn) ** 2, axis=-1) + 1e-12)
        return (2.0 - d) * 0.5

    ga = jax.nn.sigmoid(_dot(up, ipa)) * _gs(g_ref[5])
    gw = jax.nn.sigmoid(_dot(up, ipw)) * _gs(g_ref[6])
    gst = jax.nn.sigmoid(_dot(up, ips)) * _gs(g_ref[7])
    out_ref[...] = jnp.stack([ga, gw, gst], axis=0)


_tc_batch = pl.pallas_call(
    _tc_batch_body,
    out_shape=jax.ShapeDtypeStruct((3, BATCH), jnp.float32),
    grid=(8,),
    in_specs=[pl.BlockSpec((8, 512, D), lambda i: (0, i, 0))],
    out_specs=pl.BlockSpec((3, 512), lambda i: (0, i)),
)


def kernel(users, adjacent_items, weak_items, strong_items, edge_index, edge_weight,
           user_preference, item_preference, user_structure, item_structure):
    dst = edge_index[0].astype(jnp.int32)
    src = edge_index[1].astype(jnp.int32)
    w = edge_weight.astype(jnp.float32)
    pad = E_PAD - N_EDGE
    src_p = jnp.concatenate([src, jnp.zeros((pad,), jnp.int32)]).reshape(NW * CH_PER_W, CHUNK)
    dst_p = jnp.concatenate([dst, jnp.zeros((pad,), jnp.int32)]).reshape(NW * CH_PER_W, CHUNK)
    w_p = jnp.concatenate([w, jnp.zeros((pad,), jnp.float32)]).reshape(NW * CH_PER_W, CHUNK)
    epk = jnp.stack([src_p, dst_p], axis=1)
    zeros = jnp.zeros((N_NODES_PAD, D), jnp.float32)

    cat = jnp.concatenate([user_preference, item_preference], axis=0)
    pref = _tc_prep(cat)
    macc = pref
    for _ in range(N_LAYER):
        part = _sc_propagate(pref, epk, w_p, zeros)[:, :N_NODES, :]
        pref, macc = _tc_post(part, macc)

    users_pref = macc[:N_USERS]
    items_pref = macc[N_USERS:]
    ui = users.astype(jnp.int32).reshape(NW, BPW)
    ai = adjacent_items.astype(jnp.int32).reshape(NW, BPW)
    wi = weak_items.astype(jnp.int32).reshape(NW, BPW)
    si = strong_items.astype(jnp.int32).reshape(NW, BPW)
    g8 = _sc_gather(users_pref, items_pref, user_structure, item_structure,
                    ui, ai, wi, si)
    return _tc_batch(g8)


# speedup vs baseline: 1.2355x; 1.0398x over previous
import functools

import jax
import jax.numpy as jnp
from jax import lax
from jax.experimental import pallas as pl
from jax.experimental.pallas import tpu as pltpu
from jax.experimental.pallas import tpu_sc as plsc

N_USERS = 5000
N_ITEMS = 5000
D = 128
N_NODES = N_USERS + N_ITEMS
N_LAYER = 3
N_EDGE = 320000
BATCH = 4096

NC = 2
NS = 16
NW = NC * NS
CHUNK = 112
CH_PER_W = 90
CH0 = 132
CH1 = 48
E_PAD = NW * CH_PER_W * CHUNK
N_NODES_PAD = 10240
ROWS_PER_SUB = N_NODES_PAD // NS
BPW = BATCH // NW

_mesh = plsc.VectorSubcoreMesh(core_axis_name="c", subcore_axis_name="s")


NROW = 3
NIDX = 6


@functools.partial(
    pl.kernel,
    out_type=jax.ShapeDtypeStruct((NC, N_NODES_PAD, D), jnp.float32),
    mesh=_mesh,
    scratch_types=(
        [pltpu.VMEM((2, CHUNK), jnp.int32) for _ in range(NIDX)]
        + [pltpu.VMEM((CHUNK,), jnp.float32) for _ in range(NIDX)]
        + [pltpu.VMEM((CHUNK, D), jnp.float32) for _ in range(NROW)]
        + [pltpu.SemaphoreType.DMA] * (NIDX + 2 * NROW)
        + [pltpu.VMEM_SHARED((N_NODES_PAD, D), jnp.float32)]
    ),
)
def _sc_propagate(pref_hbm, epk_hbm, w_hbm, zeros_hbm, out_hbm, *refs):
    ebuf = refs[0:NIDX]
    wbuf = refs[NIDX:2 * NIDX]
    rows = refs[2 * NIDX:2 * NIDX + NROW]
    isem = refs[2 * NIDX + NROW:3 * NIDX + NROW]
    gsem = refs[3 * NIDX + NROW:3 * NIDX + 2 * NROW]
    ssem = refs[3 * NIDX + 2 * NROW:3 * NIDX + 3 * NROW]
    acc_sh = refs[3 * NIDX + 3 * NROW]
    c = lax.axis_index("c")
    s = lax.axis_index("s")
    ch_c = jnp.where(c == 0, CH0, CH1)
    base = jnp.where(c == 0, s * CH0, NS * CH0 + s * CH1)
    pltpu.sync_copy(zeros_hbm.at[pl.ds(s * ROWS_PER_SUB, ROWS_PER_SUB)],
                    acc_sh.at[pl.ds(s * ROWS_PER_SUB, ROWS_PER_SUB)])
    plsc.subcore_barrier()

    def _issue_idx(ci, eb):
        pltpu.async_copy(epk_hbm.at[base + ci], ebuf[eb], isem[eb])
        pltpu.async_copy(w_hbm.at[base + ci], wbuf[eb], isem[eb])

    def _wait_idx(eb):
        pltpu.make_async_copy(epk_hbm.at[base], ebuf[eb], isem[eb]).wait()
        pltpu.make_async_copy(w_hbm.at[base], wbuf[eb], isem[eb]).wait()

    def _issue_gather(eb, r):
        pltpu.async_copy(pref_hbm.at[ebuf[eb].at[0]], rows[r], gsem[r])

    def _wait_gather(eb, r):
        pltpu.make_async_copy(pref_hbm.at[ebuf[eb].at[0]], rows[r], gsem[r]).wait()

    def _issue_scatter(eb, r):
        pltpu.async_copy(rows[r], acc_sh.at[ebuf[eb].at[1]], ssem[r], add=True)

    def _wait_scatter(eb, r):
        pltpu.make_async_copy(rows[r], acc_sh.at[ebuf[eb].at[1]], ssem[r]).wait()

    def _scale(eb, r):
        def grp(g, carry):
            wreg = wbuf[eb][pl.ds(g * 16, 16)]
            for i in range(16):
                wb = jnp.full((16,), wreg[i], jnp.float32)
                e = g * 16 + i
                for j in range(D // 16):
                    rows[r][e, pl.ds(j * 16, 16)] = rows[r][e, pl.ds(j * 16, 16)] * wb
            return carry
        lax.fori_loop(0, CHUNK // 16, grp, 0)

    _issue_idx(0, 0)
    _issue_idx(1, 1)
    _issue_idx(2, 2)
    _wait_idx(0)
    _issue_gather(0, 0)

    def outer(k, carry):
        for u in range(NIDX):
            ci = NIDX * k + u
            r = u % NROW

            @pl.when(ci >= 2)
            def _():
                _wait_scatter((u + 4) % NIDX, (u + 1) % NROW)

            @pl.when(ci + 1 < ch_c)
            def _():
                _wait_idx((u + 1) % NIDX)
                _issue_gather((u + 1) % NIDX, (u + 1) % NROW)

            @pl.when(ci + 3 < ch_c)
            def _():
                _issue_idx(ci + 3, (u + 3) % NIDX)

            _wait_gather(u, r)
            _scale(u, r)
            _issue_scatter(u, r)
        return carry

    lax.fori_loop(0, ch_c // NIDX, outer, 0)
    _wait_scatter(4, 1)
    _wait_scatter(5, 2)
    plsc.subcore_barrier()
    pltpu.sync_copy(acc_sh.at[pl.ds(s * ROWS_PER_SUB, ROWS_PER_SUB)],
                    out_hbm.at[c, pl.ds(s * ROWS_PER_SUB, ROWS_PER_SUB)])


def _leaky_norm(x):
    y = jnp.where(x >= 0, x, 0.1 * x)
    n = jnp.sqrt(jnp.sum(y * y, axis=-1, keepdims=True))
    return y / jnp.maximum(n, 1e-12)


def _tc_prep_body(cat_ref, out_ref):
    out_ref[...] = _leaky_norm(cat_ref[...])


_tc_prep = pl.pallas_call(
    _tc_prep_body,
    out_shape=jax.ShapeDtypeStruct((N_NODES, D), jnp.float32),
    grid=(10,),
    in_specs=[pl.BlockSpec((1000, D), lambda i: (i, 0))],
    out_specs=pl.BlockSpec((1000, D), lambda i: (i, 0)),
)


def _tc_post_body(part_ref, macc_ref, pref_out, macc_out):
    p = _leaky_norm(part_ref[0] + part_ref[1])
    pref_out[...] = p
    macc_out[...] = macc_ref[...] + p


_tc_post = pl.pallas_call(
    _tc_post_body,
    out_shape=[jax.ShapeDtypeStruct((N_NODES, D), jnp.float32)] * 2,
    grid=(10,),
    in_specs=[pl.BlockSpec((NC, 1000, D), lambda i: (0, i, 0)),
              pl.BlockSpec((1000, D), lambda i: (i, 0))],
    out_specs=[pl.BlockSpec((1000, D), lambda i: (i, 0))] * 2,
)


@functools.partial(
    pl.kernel,
    out_type=jax.ShapeDtypeStruct((8, BATCH, D), jnp.float32),
    mesh=_mesh,
    scratch_types=[
        pltpu.VMEM((BPW,), jnp.int32),
        pltpu.VMEM((BPW, D), jnp.float32),
        pltpu.SemaphoreType.DMA,
    ],
)
def _sc_gather(upref, ipref, ustr, istr, users_i, adj_i, weak_i, strong_i,
               out_hbm, idx_v, rows_v, sem):
    c = lax.axis_index("c")
    s = lax.axis_index("s")
    wid = c * NS + s
    tasks = ((upref, users_i), (ipref, adj_i), (ipref, weak_i), (ipref, strong_i),
             (ustr, users_i), (istr, adj_i), (istr, weak_i), (istr, strong_i))
    for t, (tab, idx) in enumerate(tasks):
        pltpu.sync_copy(idx.at[wid], idx_v)
        pltpu.async_copy(tab.at[idx_v], rows_v, sem).wait()
        pltpu.sync_copy(rows_v, out_hbm.at[t, pl.ds(wid * BPW, BPW)])


def _tc_batch_body(g_ref, out_ref):
    up = g_ref[0] * 0.25
    ipa = g_ref[1] * 0.25
    ipw = g_ref[2] * 0.25
    ips = g_ref[3] * 0.25
    us = g_ref[4]

    def _dot(a, b):
        return jnp.sum(a * b, axis=-1)

    def _norm(x):
        n = jnp.sqrt(jnp.sum(x * x, axis=-1, keepdims=True))
        return x / jnp.maximum(n, 1e-12)

    usn = _norm(us)

    def _gs(im):
        imn = _norm(im)
        d = jnp.sqrt(jnp.sum((usn - imn) ** 2, axis=-1) + 1e-12)
        return (2.0 - d) * 0.5

    ga = jax.nn.sigmoid(_dot(up, ipa)) * _gs(g_ref[5])
    gw = jax.nn.sigmoid(_dot(up, ipw)) * _gs(g_ref[6])
    gst = jax.nn.sigmoid(_dot(up, ips)) * _gs(g_ref[7])
    out_ref[...] = jnp.stack([ga, gw, gst], axis=0)


_tc_batch = pl.pallas_call(
    _tc_batch_body,
    out_shape=jax.ShapeDtypeStruct((3, BATCH), jnp.float32),
    grid=(8,),
    in_specs=[pl.BlockSpec((8, 512, D), lambda i: (0, i, 0))],
    out_specs=pl.BlockSpec((3, 512), lambda i: (0, i)),
)


def kernel(users, adjacent_items, weak_items, strong_items, edge_index, edge_weight,
           user_preference, item_preference, user_structure, item_structure):
    dst = edge_index[0].astype(jnp.int32)
    src = edge_index[1].astype(jnp.int32)
    w = edge_weight.astype(jnp.float32)
    pad = E_PAD - N_EDGE
    src_p = jnp.concatenate([src, jnp.zeros((pad,), jnp.int32)]).reshape(NW * CH_PER_W, CHUNK)
    dst_p = jnp.concatenate([dst, jnp.zeros((pad,), jnp.int32)]).reshape(NW * CH_PER_W, CHUNK)
    w_p = jnp.concatenate([w, jnp.zeros((pad,), jnp.float32)]).reshape(NW * CH_PER_W, CHUNK)
    epk = jnp.stack([src_p, dst_p], axis=1)
    zeros = jnp.zeros((N_NODES_PAD, D), jnp.float32)

    cat = jnp.concatenate([user_preference, item_preference], axis=0)
    pref = _tc_prep(cat)
    macc = pref
    for _ in range(N_LAYER):
        part = _sc_propagate(pref, epk, w_p, zeros)[:, :N_NODES, :]
        pref, macc = _tc_post(part, macc)

    users_pref = macc[:N_USERS]
    items_pref = macc[N_USERS:]
    ui = users.astype(jnp.int32).reshape(NW, BPW)
    ai = adjacent_items.astype(jnp.int32).reshape(NW, BPW)
    wi = weak_items.astype(jnp.int32).reshape(NW, BPW)
    si = strong_items.astype(jnp.int32).reshape(NW, BPW)
    g8 = _sc_gather(users_pref, items_pref, user_structure, item_structure,
                    ui, ai, wi, si)
    return _tc_batch(g8)


# speedup vs baseline: 1.2620x; 1.0215x over previous
import functools

import jax
import jax.numpy as jnp
from jax import lax
from jax.experimental import pallas as pl
from jax.experimental.pallas import tpu as pltpu
from jax.experimental.pallas import tpu_sc as plsc

N_USERS = 5000
N_ITEMS = 5000
D = 128
N_NODES = N_USERS + N_ITEMS
N_LAYER = 3
N_EDGE = 320000
BATCH = 4096

NC = 2
NS = 16
NW = NC * NS
CHUNK = 112
CH_PER_W = 90
CH0 = 138
CH1 = 42
E_PAD = NW * CH_PER_W * CHUNK
N_NODES_PAD = 10240
ROWS_PER_SUB = N_NODES_PAD // NS
BPW = BATCH // NW

_mesh = plsc.VectorSubcoreMesh(core_axis_name="c", subcore_axis_name="s")


NROW = 3
NIDX = 6


@functools.partial(
    pl.kernel,
    out_type=jax.ShapeDtypeStruct((NC, N_NODES_PAD, D), jnp.float32),
    mesh=_mesh,
    scratch_types=(
        [pltpu.VMEM((2, CHUNK), jnp.int32) for _ in range(NIDX)]
        + [pltpu.VMEM((CHUNK,), jnp.float32) for _ in range(NIDX)]
        + [pltpu.VMEM((CHUNK, D), jnp.float32) for _ in range(NROW)]
        + [pltpu.SemaphoreType.DMA] * (NIDX + 2 * NROW)
        + [pltpu.VMEM_SHARED((N_NODES_PAD, D), jnp.float32)]
    ),
)
def _sc_propagate(pref_hbm, epk_hbm, w_hbm, zeros_hbm, out_hbm, *refs):
    ebuf = refs[0:NIDX]
    wbuf = refs[NIDX:2 * NIDX]
    rows = refs[2 * NIDX:2 * NIDX + NROW]
    isem = refs[2 * NIDX + NROW:3 * NIDX + NROW]
    gsem = refs[3 * NIDX + NROW:3 * NIDX + 2 * NROW]
    ssem = refs[3 * NIDX + 2 * NROW:3 * NIDX + 3 * NROW]
    acc_sh = refs[3 * NIDX + 3 * NROW]
    c = lax.axis_index("c")
    s = lax.axis_index("s")
    ch_c = jnp.where(c == 0, CH0, CH1)
    base = jnp.where(c == 0, s * CH0, NS * CH0 + s * CH1)
    pltpu.sync_copy(zeros_hbm.at[pl.ds(s * ROWS_PER_SUB, ROWS_PER_SUB)],
                    acc_sh.at[pl.ds(s * ROWS_PER_SUB, ROWS_PER_SUB)])
    plsc.subcore_barrier()

    def _issue_idx(ci, eb):
        pltpu.async_copy(epk_hbm.at[base + ci], ebuf[eb], isem[eb])
        pltpu.async_copy(w_hbm.at[base + ci], wbuf[eb], isem[eb])

    def _wait_idx(eb):
        pltpu.make_async_copy(epk_hbm.at[base], ebuf[eb], isem[eb]).wait()
        pltpu.make_async_copy(w_hbm.at[base], wbuf[eb], isem[eb]).wait()

    def _issue_gather(eb, r):
        pltpu.async_copy(pref_hbm.at[ebuf[eb].at[0]], rows[r], gsem[r])

    def _wait_gather(eb, r):
        pltpu.make_async_copy(pref_hbm.at[ebuf[eb].at[0]], rows[r], gsem[r]).wait()

    def _issue_scatter(eb, r):
        pltpu.async_copy(rows[r], acc_sh.at[ebuf[eb].at[1]], ssem[r], add=True)

    def _wait_scatter(eb, r):
        pltpu.make_async_copy(rows[r], acc_sh.at[ebuf[eb].at[1]], ssem[r]).wait()

    def _scale(eb, r):
        def grp(g, carry):
            wreg = wbuf[eb][pl.ds(g * 16, 16)]
            for i in range(16):
                wb = jnp.full((16,), wreg[i], jnp.float32)
                e = g * 16 + i
                for j in range(D // 16):
                    rows[r][e, pl.ds(j * 16, 16)] = rows[r][e, pl.ds(j * 16, 16)] * wb
            return carry
        lax.fori_loop(0, CHUNK // 16, grp, 0)

    _issue_idx(0, 0)
    _issue_idx(1, 1)
    _issue_idx(2, 2)
    _wait_idx(0)
    _issue_gather(0, 0)

    def outer(k, carry):
        for u in range(NIDX):
            ci = NIDX * k + u
            r = u % NROW

            @pl.when(ci >= 2)
            def _():
                _wait_scatter((u + 4) % NIDX, (u + 1) % NROW)

            @pl.when(ci + 1 < ch_c)
            def _():
                _wait_idx((u + 1) % NIDX)
                _issue_gather((u + 1) % NIDX, (u + 1) % NROW)

            @pl.when(ci + 3 < ch_c)
            def _():
                _issue_idx(ci + 3, (u + 3) % NIDX)

            _wait_gather(u, r)
            _scale(u, r)
            _issue_scatter(u, r)
        return carry

    lax.fori_loop(0, ch_c // NIDX, outer, 0)
    _wait_scatter(4, 1)
    _wait_scatter(5, 2)
    plsc.subcore_barrier()
    pltpu.sync_copy(acc_sh.at[pl.ds(s * ROWS_PER_SUB, ROWS_PER_SUB)],
                    out_hbm.at[c, pl.ds(s * ROWS_PER_SUB, ROWS_PER_SUB)])


def _leaky_norm(x):
    y = jnp.where(x >= 0, x, 0.1 * x)
    n = jnp.sqrt(jnp.sum(y * y, axis=-1, keepdims=True))
    return y / jnp.maximum(n, 1e-12)


def _tc_prep_body(cat_ref, out_ref):
    out_ref[...] = _leaky_norm(cat_ref[...])


_tc_prep = pl.pallas_call(
    _tc_prep_body,
    out_shape=jax.ShapeDtypeStruct((N_NODES, D), jnp.float32),
    grid=(10,),
    in_specs=[pl.BlockSpec((1000, D), lambda i: (i, 0))],
    out_specs=pl.BlockSpec((1000, D), lambda i: (i, 0)),
)


def _tc_post_body(part_ref, macc_ref, pref_out, macc_out):
    p = _leaky_norm(part_ref[0] + part_ref[1])
    pref_out[...] = p
    macc_out[...] = macc_ref[...] + p


_tc_post = pl.pallas_call(
    _tc_post_body,
    out_shape=[jax.ShapeDtypeStruct((N_NODES, D), jnp.float32)] * 2,
    grid=(10,),
    in_specs=[pl.BlockSpec((NC, 1000, D), lambda i: (0, i, 0)),
              pl.BlockSpec((1000, D), lambda i: (i, 0))],
    out_specs=[pl.BlockSpec((1000, D), lambda i: (i, 0))] * 2,
)


@functools.partial(
    pl.kernel,
    out_type=jax.ShapeDtypeStruct((8, BATCH, D), jnp.float32),
    mesh=_mesh,
    scratch_types=[
        pltpu.VMEM((BPW,), jnp.int32),
        pltpu.VMEM((BPW, D), jnp.float32),
        pltpu.SemaphoreType.DMA,
    ],
)
def _sc_gather(upref, ipref, ustr, istr, users_i, adj_i, weak_i, strong_i,
               out_hbm, idx_v, rows_v, sem):
    c = lax.axis_index("c")
    s = lax.axis_index("s")
    wid = c * NS + s
    tasks = ((upref, users_i), (ipref, adj_i), (ipref, weak_i), (ipref, strong_i),
             (ustr, users_i), (istr, adj_i), (istr, weak_i), (istr, strong_i))
    for t, (tab, idx) in enumerate(tasks):
        pltpu.sync_copy(idx.at[wid], idx_v)
        pltpu.async_copy(tab.at[idx_v], rows_v, sem).wait()
        pltpu.sync_copy(rows_v, out_hbm.at[t, pl.ds(wid * BPW, BPW)])


def _tc_batch_body(g_ref, out_ref):
    up = g_ref[0] * 0.25
    ipa = g_ref[1] * 0.25
    ipw = g_ref[2] * 0.25
    ips = g_ref[3] * 0.25
    us = g_ref[4]

    def _dot(a, b):
        return jnp.sum(a * b, axis=-1)

    def _norm(x):
        n = jnp.sqrt(jnp.sum(x * x, axis=-1, keepdims=True))
        return x / jnp.maximum(n, 1e-12)

    usn = _norm(us)

    def _gs(im):
        imn = _norm(im)
        d = jnp.sqrt(jnp.sum((usn - imn) ** 2, axis=-1) + 1e-12)
        return (2.0 - d) * 0.5

    ga = jax.nn.sigmoid(_dot(up, ipa)) * _gs(g_ref[5])
    gw = jax.nn.sigmoid(_dot(up, ipw)) * _gs(g_ref[6])
    gst = jax.nn.sigmoid(_dot(up, ips)) * _gs(g_ref[7])
    out_ref[...] = jnp.stack([ga, gw, gst], axis=0)


_tc_batch = pl.pallas_call(
    _tc_batch_body,
    out_shape=jax.ShapeDtypeStruct((3, BATCH), jnp.float32),
    grid=(8,),
    in_specs=[pl.BlockSpec((8, 512, D), lambda i: (0, i, 0))],
    out_specs=pl.BlockSpec((3, 512), lambda i: (0, i)),
)


def kernel(users, adjacent_items, weak_items, strong_items, edge_index, edge_weight,
           user_preference, item_preference, user_structure, item_structure):
    dst = edge_index[0].astype(jnp.int32)
    src = edge_index[1].astype(jnp.int32)
    w = edge_weight.astype(jnp.float32)
    pad = E_PAD - N_EDGE
    src_p = jnp.concatenate([src, jnp.zeros((pad,), jnp.int32)]).reshape(NW * CH_PER_W, CHUNK)
    dst_p = jnp.concatenate([dst, jnp.zeros((pad,), jnp.int32)]).reshape(NW * CH_PER_W, CHUNK)
    w_p = jnp.concatenate([w, jnp.zeros((pad,), jnp.float32)]).reshape(NW * CH_PER_W, CHUNK)
    epk = jnp.stack([src_p, dst_p], axis=1)
    zeros = jnp.zeros((N_NODES_PAD, D), jnp.float32)

    cat = jnp.concatenate([user_preference, item_preference], axis=0)
    pref = _tc_prep(cat)
    macc = pref
    for _ in range(N_LAYER):
        part = _sc_propagate(pref, epk, w_p, zeros)[:, :N_NODES, :]
        pref, macc = _tc_post(part, macc)

    users_pref = macc[:N_USERS]
    items_pref = macc[N_USERS:]
    ui = users.astype(jnp.int32).reshape(NW, BPW)
    ai = adjacent_items.astype(jnp.int32).reshape(NW, BPW)
    wi = weak_items.astype(jnp.int32).reshape(NW, BPW)
    si = strong_items.astype(jnp.int32).reshape(NW, BPW)
    g8 = _sc_gather(users_pref, items_pref, user_structure, item_structure,
                    ui, ai, wi, si)
    return _tc_batch(g8)
